# baseline (speedup 1.0000x reference)
import functools

import jax
import jax.numpy as jnp
from jax import lax
from jax.experimental import pallas as pl
from jax.experimental.pallas import tpu as pltpu
from jax.experimental.pallas import tpu_sc as plsc

DIM = 192
NUM_HEADS = 6
GROUP = 128
QKV_PAD = 640
OUT_PAD = 256
_NC = 2
_NS = 16
_NW = _NC * _NS


@functools.lru_cache(maxsize=None)
def _make_sc_row_gather(rows: int, d: int, chunk: int):
    assert rows % (_NW * chunk) == 0 and d % 128 == 0
    per_worker = rows // _NW
    n_chunks = per_worker // chunk
    mesh = plsc.VectorSubcoreMesh(core_axis_name="c", subcore_axis_name="s")

    @functools.partial(
        pl.kernel,
        out_type=jax.ShapeDtypeStruct((rows, d), jnp.float32),
        mesh=mesh,
        scratch_types=[
            pltpu.VMEM((chunk,), jnp.int32),
            pltpu.VMEM((chunk, d), jnp.float32),
            pltpu.SemaphoreType.DMA,
        ],
    )
    def gather_kernel(table_hbm, idx_hbm, out_hbm, idx_v, rows_v, sem):
        wid = lax.axis_index("s") * _NC + lax.axis_index("c")
        base = wid * per_worker
        for ch in range(n_chunks):
            off = base + ch * chunk
            pltpu.sync_copy(idx_hbm.at[pl.ds(off, chunk)], idx_v)
            pltpu.async_copy(table_hbm.at[idx_v], rows_v, sem).wait()
            pltpu.sync_copy(rows_v, out_hbm.at[pl.ds(off, chunk)])

    return gather_kernel


@functools.lru_cache(maxsize=None)
def _make_sc_row_scatter(rows: int, d: int, chunk: int):
    assert rows % (_NW * chunk) == 0 and d % 128 == 0
    per_worker = rows // _NW
    n_chunks = per_worker // chunk
    mesh = plsc.VectorSubcoreMesh(core_axis_name="c", subcore_axis_name="s")

    @functools.partial(
        pl.kernel,
        out_type=jax.ShapeDtypeStruct((rows, d), jnp.float32),
        mesh=mesh,
        scratch_types=[
            pltpu.VMEM((chunk,), jnp.int32),
            pltpu.VMEM((chunk, d), jnp.float32),
            pltpu.SemaphoreType.DMA,
        ],
    )
    def scatter_kernel(src_hbm, idx_hbm, out_hbm, idx_v, rows_v, sem):
        wid = lax.axis_index("s") * _NC + lax.axis_index("c")
        base = wid * per_worker
        for ch in range(n_chunks):
            off = base + ch * chunk
            pltpu.sync_copy(idx_hbm.at[pl.ds(off, chunk)], idx_v)
            pltpu.sync_copy(src_hbm.at[pl.ds(off, chunk)], rows_v)
            pltpu.async_copy(rows_v, out_hbm.at[idx_v], sem).wait()

    return scatter_kernel


def _attn_body(scale, gpb, x_ref, wt_ref, b_ref, o_ref):
    dh = DIM // NUM_HEADS
    wt = wt_ref[...]
    bias = b_ref[...]
    for g in range(gpb):
        x = x_ref[g]
        q = x[:, :DIM] * scale
        k = x[:, DIM:2 * DIM]
        v = x[:, 2 * DIM:3 * DIM]
        y = bias
        for h in range(NUM_HEADS):
            sl = slice(h * dh, (h + 1) * dh)
            qh = q[:, sl]
            kh = k[:, sl]
            vh = v[:, sl]
            s = lax.dot_general(qh, kh, (((1,), (1,)), ((), ())),
                                preferred_element_type=jnp.float32)
            e = jnp.exp(jnp.minimum(s, 80.0))
            acc = jnp.dot(e, vh, preferred_element_type=jnp.float32)
            r = lax.reciprocal(jnp.sum(e, axis=-1, keepdims=True))
            y = y + jnp.dot(acc * r, wt[sl, :],
                            preferred_element_type=jnp.float32)
        o_ref[g, :, :DIM] = y


def _tc_attention(shuf, w_t, bias2d, scale, gpb=8):
    n_groups = shuf.shape[0]
    return pl.pallas_call(
        functools.partial(_attn_body, scale, gpb),
        grid=(n_groups // gpb,),
        in_specs=[
            pl.BlockSpec((gpb, GROUP, QKV_PAD), lambda i: (i, 0, 0)),
            pl.BlockSpec((DIM, DIM), lambda i: (0, 0)),
            pl.BlockSpec((1, DIM), lambda i: (0, 0)),
        ],
        out_specs=pl.BlockSpec((gpb, GROUP, OUT_PAD), lambda i: (i, 0, 0)),
        out_shape=jax.ShapeDtypeStruct((n_groups, GROUP, OUT_PAD),
                                       jnp.float32),
    )(shuf, w_t, bias2d)


def kernel(qkv, tk_id, x_size, proj_w, proj_b):
    b, n, c3 = qkv.shape
    c = c3 // 3
    scale = (c // NUM_HEADS) ** (-0.5)
    ng = n // GROUP

    sort_idx = jnp.argsort(tk_id, axis=-1)

    offs = (jnp.arange(b, dtype=jnp.int32) * n)[:, None]
    sidx = (sort_idx.astype(jnp.int32) + offs).reshape(b * n)

    qkv_pad = jnp.pad(qkv.reshape(b * n, c3), ((0, 0), (0, QKV_PAD - c3)))
    shuf = _make_sc_row_gather(b * n, QKV_PAD, 128)(qkv_pad, sidx)

    y = _tc_attention(shuf.reshape(b * ng, GROUP, QKV_PAD), proj_w.T,
                      proj_b.reshape(1, c), scale)

    xp = _make_sc_row_scatter(b * n, OUT_PAD, 128)(
        y.reshape(b * n, OUT_PAD), sidx)
    return xp[:, :c].reshape(b, n, c)


# speedup vs baseline: 1.0265x; 1.0265x over previous
import functools

import jax
import jax.numpy as jnp
from jax import lax
from jax.experimental import pallas as pl
from jax.experimental.pallas import tpu as pltpu
from jax.experimental.pallas import tpu_sc as plsc

DIM = 192
NUM_HEADS = 6
GROUP = 128
QKV_PAD = 640
OUT_PAD = 256
_NC = 2
_NS = 16
_NW = _NC * _NS


@functools.lru_cache(maxsize=None)
def _make_sc_row_gather(rows: int, d: int, chunk: int):
    assert rows % (_NW * chunk) == 0 and d % 128 == 0
    per_worker = rows // _NW
    n_chunks = per_worker // chunk
    assert n_chunks >= 2
    mesh = plsc.VectorSubcoreMesh(core_axis_name="c", subcore_axis_name="s")

    @functools.partial(
        pl.kernel,
        out_type=jax.ShapeDtypeStruct((rows, d), jnp.float32),
        mesh=mesh,
        scratch_types=[
            pltpu.VMEM((chunk,), jnp.int32),
            pltpu.VMEM((chunk,), jnp.int32),
            pltpu.VMEM((chunk, d), jnp.float32),
            pltpu.VMEM((chunk, d), jnp.float32),
            pltpu.SemaphoreType.DMA,
            pltpu.SemaphoreType.DMA,
            pltpu.SemaphoreType.DMA,
            pltpu.SemaphoreType.DMA,
        ],
    )
    def gather_kernel(table_hbm, idx_hbm, out_hbm, idx0, idx1, rows0, rows1,
                      sg0, sg1, ss0, ss1):
        wid = lax.axis_index("s") * _NC + lax.axis_index("c")
        base = wid * per_worker
        bufs = [(idx0, rows0, sg0, ss0), (idx1, rows1, sg1, ss1)]
        store_h = [None, None]
        for ch in range(n_chunks):
            idx_v, rows_v, sg, ss = bufs[ch % 2]
            off = base + ch * chunk
            if store_h[ch % 2] is not None:
                store_h[ch % 2].wait()
            pltpu.sync_copy(idx_hbm.at[pl.ds(off, chunk)], idx_v)
            pltpu.async_copy(table_hbm.at[idx_v], rows_v, sg).wait()
            store_h[ch % 2] = pltpu.async_copy(
                rows_v, out_hbm.at[pl.ds(off, chunk)], ss)
        store_h[0].wait()
        store_h[1].wait()

    return gather_kernel


@functools.lru_cache(maxsize=None)
def _make_sc_row_scatter(rows: int, d: int, chunk: int):
    assert rows % (_NW * chunk) == 0 and d % 128 == 0
    per_worker = rows // _NW
    n_chunks = per_worker // chunk
    mesh = plsc.VectorSubcoreMesh(core_axis_name="c", subcore_axis_name="s")

    @functools.partial(
        pl.kernel,
        out_type=jax.ShapeDtypeStruct((rows, d), jnp.float32),
        mesh=mesh,
        scratch_types=[
            pltpu.VMEM((chunk,), jnp.int32),
            pltpu.VMEM((chunk,), jnp.int32),
            pltpu.VMEM((chunk, d), jnp.float32),
            pltpu.VMEM((chunk, d), jnp.float32),
            pltpu.SemaphoreType.DMA,
            pltpu.SemaphoreType.DMA,
            pltpu.SemaphoreType.DMA,
            pltpu.SemaphoreType.DMA,
        ],
    )
    def scatter_kernel(src_hbm, idx_hbm, out_hbm, idx0, idx1, rows0, rows1,
                       sl0, sl1, ss0, ss1):
        wid = lax.axis_index("s") * _NC + lax.axis_index("c")
        base = wid * per_worker
        bufs = [(idx0, rows0, sl0, ss0), (idx1, rows1, sl1, ss1)]
        scat_h = [None, None]
        for ch in range(n_chunks):
            idx_v, rows_v, sl_, ss = bufs[ch % 2]
            off = base + ch * chunk
            if scat_h[ch % 2] is not None:
                scat_h[ch % 2].wait()
            pltpu.sync_copy(idx_hbm.at[pl.ds(off, chunk)], idx_v)
            pltpu.async_copy(src_hbm.at[pl.ds(off, chunk)], rows_v, sl_).wait()
            scat_h[ch % 2] = pltpu.async_copy(rows_v, out_hbm.at[idx_v], ss)
        scat_h[0].wait()
        scat_h[1].wait()

    return scatter_kernel


def _attn_body(scale, gpb, x_ref, wt_ref, b_ref, o_ref):
    dh = DIM // NUM_HEADS
    wt = wt_ref[...]
    bias = b_ref[...]
    for g in range(gpb):
        x = x_ref[g]
        q = x[:, :DIM] * scale
        k = x[:, DIM:2 * DIM]
        v = x[:, 2 * DIM:3 * DIM]
        y = bias
        for h in range(NUM_HEADS):
            sl = slice(h * dh, (h + 1) * dh)
            qh = q[:, sl]
            kh = k[:, sl]
            vh = v[:, sl]
            s = lax.dot_general(qh, kh, (((1,), (1,)), ((), ())),
                                preferred_element_type=jnp.float32)
            e = jnp.exp(jnp.minimum(s, 80.0))
            acc = jnp.dot(e, vh, preferred_element_type=jnp.float32)
            r = lax.reciprocal(jnp.sum(e, axis=-1, keepdims=True))
            y = y + jnp.dot(acc * r, wt[sl, :],
                            preferred_element_type=jnp.float32)
        o_ref[g, :, :DIM] = y


def _tc_attention(shuf, w_t, bias2d, scale, gpb=4):
    n_groups = shuf.shape[0]
    return pl.pallas_call(
        functools.partial(_attn_body, scale, gpb),
        grid=(n_groups // gpb,),
        in_specs=[
            pl.BlockSpec((gpb, GROUP, QKV_PAD), lambda i: (i, 0, 0)),
            pl.BlockSpec((DIM, DIM), lambda i: (0, 0)),
            pl.BlockSpec((1, DIM), lambda i: (0, 0)),
        ],
        out_specs=pl.BlockSpec((gpb, GROUP, OUT_PAD), lambda i: (i, 0, 0)),
        out_shape=jax.ShapeDtypeStruct((n_groups, GROUP, OUT_PAD),
                                       jnp.float32),
    )(shuf, w_t, bias2d)


def kernel(qkv, tk_id, x_size, proj_w, proj_b):
    b, n, c3 = qkv.shape
    c = c3 // 3
    scale = (c // NUM_HEADS) ** (-0.5)
    ng = n // GROUP

    sort_idx = jnp.argsort(tk_id, axis=-1)

    offs = (jnp.arange(b, dtype=jnp.int32) * n)[:, None]
    sidx = (sort_idx.astype(jnp.int32) + offs).reshape(b * n)

    qkv_pad = jnp.pad(qkv.reshape(b * n, c3), ((0, 0), (0, QKV_PAD - c3)))
    shuf = _make_sc_row_gather(b * n, QKV_PAD, 64)(qkv_pad, sidx)

    y = _tc_attention(shuf.reshape(b * ng, GROUP, QKV_PAD), proj_w.T,
                      proj_b.reshape(1, c), scale)

    xp = _make_sc_row_scatter(b * n, OUT_PAD, 128)(
        y.reshape(b * n, OUT_PAD), sidx)
    return xp[:, :c].reshape(b, n, c)


# speedup vs baseline: 1.0958x; 1.0676x over previous
import functools

import jax
import jax.numpy as jnp
from jax import lax
from jax.experimental import pallas as pl
from jax.experimental.pallas import tpu as pltpu
from jax.experimental.pallas import tpu_sc as plsc

DIM = 192
NUM_HEADS = 6
GROUP = 128
QKV_PAD = 640
OUT_PAD = 256
_NC = 2
_NS = 16
_NW = _NC * _NS


@functools.lru_cache(maxsize=None)
def _make_sc_row_gather(rows: int, d: int, chunk: int):
    assert rows % (_NW * chunk) == 0 and d % 128 == 0
    per_worker = rows // _NW
    n_chunks = per_worker // chunk
    assert n_chunks >= 2
    mesh = plsc.VectorSubcoreMesh(core_axis_name="c", subcore_axis_name="s")

    @functools.partial(
        pl.kernel,
        out_type=jax.ShapeDtypeStruct((rows, d), jnp.float32),
        mesh=mesh,
        scratch_types=[
            pltpu.VMEM((chunk,), jnp.int32),
            pltpu.VMEM((chunk,), jnp.int32),
            pltpu.VMEM((chunk, d), jnp.float32),
            pltpu.VMEM((chunk, d), jnp.float32),
            pltpu.SemaphoreType.DMA,
            pltpu.SemaphoreType.DMA,
            pltpu.SemaphoreType.DMA,
            pltpu.SemaphoreType.DMA,
        ],
    )
    def gather_kernel(table_hbm, idx_hbm, out_hbm, idx0, idx1, rows0, rows1,
                      sg0, sg1, ss0, ss1):
        wid = lax.axis_index("s") * _NC + lax.axis_index("c")
        base = wid * per_worker
        bufs = [(idx0, rows0, sg0, ss0), (idx1, rows1, sg1, ss1)]
        store_h = [None, None]
        for ch in range(n_chunks):
            idx_v, rows_v, sg, ss = bufs[ch % 2]
            off = base + ch * chunk
            if store_h[ch % 2] is not None:
                store_h[ch % 2].wait()
            pltpu.sync_copy(idx_hbm.at[pl.ds(off, chunk)], idx_v)
            pltpu.async_copy(table_hbm.at[idx_v], rows_v, sg).wait()
            store_h[ch % 2] = pltpu.async_copy(
                rows_v, out_hbm.at[pl.ds(off, chunk)], ss)
        store_h[0].wait()
        store_h[1].wait()

    return gather_kernel


@functools.lru_cache(maxsize=None)
def _make_sc_row_scatter(rows: int, d: int, chunk: int):
    assert rows % (_NW * chunk) == 0 and d % 128 == 0
    per_worker = rows // _NW
    n_chunks = per_worker // chunk
    mesh = plsc.VectorSubcoreMesh(core_axis_name="c", subcore_axis_name="s")

    @functools.partial(
        pl.kernel,
        out_type=jax.ShapeDtypeStruct((rows, d), jnp.float32),
        mesh=mesh,
        scratch_types=[
            pltpu.VMEM((chunk,), jnp.int32),
            pltpu.VMEM((chunk,), jnp.int32),
            pltpu.VMEM((chunk, d), jnp.float32),
            pltpu.VMEM((chunk, d), jnp.float32),
            pltpu.SemaphoreType.DMA,
            pltpu.SemaphoreType.DMA,
            pltpu.SemaphoreType.DMA,
            pltpu.SemaphoreType.DMA,
        ],
    )
    def scatter_kernel(src_hbm, idx_hbm, out_hbm, idx0, idx1, rows0, rows1,
                       sl0, sl1, ss0, ss1):
        wid = lax.axis_index("s") * _NC + lax.axis_index("c")
        base = wid * per_worker
        bufs = [(idx0, rows0, sl0, ss0), (idx1, rows1, sl1, ss1)]
        scat_h = [None, None]
        for ch in range(n_chunks):
            idx_v, rows_v, sl_, ss = bufs[ch % 2]
            off = base + ch * chunk
            if scat_h[ch % 2] is not None:
                scat_h[ch % 2].wait()
            pltpu.sync_copy(idx_hbm.at[pl.ds(off, chunk)], idx_v)
            pltpu.async_copy(src_hbm.at[pl.ds(off, chunk)], rows_v, sl_).wait()
            scat_h[ch % 2] = pltpu.async_copy(rows_v, out_hbm.at[idx_v], ss)
        scat_h[0].wait()
        scat_h[1].wait()

    return scatter_kernel


def _attn_body(scale, gpb, x_ref, wt_ref, b_ref, o_ref):
    dh = DIM // NUM_HEADS
    wt = wt_ref[...]
    bias = b_ref[...]
    for g in range(gpb):
        x = x_ref[g]
        q = x[:, :DIM] * scale
        k = x[:, DIM:2 * DIM]
        v = x[:, 2 * DIM:3 * DIM]
        y = bias
        for h in range(NUM_HEADS):
            sl = slice(h * dh, (h + 1) * dh)
            qh = q[:, sl]
            kh = k[:, sl]
            vh = v[:, sl]
            s = lax.dot_general(qh, kh, (((1,), (1,)), ((), ())),
                                preferred_element_type=jnp.float32)
            e = jnp.exp(jnp.minimum(s, 80.0))
            acc = jnp.dot(e, vh, preferred_element_type=jnp.float32)
            r = lax.reciprocal(jnp.sum(e, axis=-1, keepdims=True))
            y = y + jnp.dot(acc * r, wt[sl, :],
                            preferred_element_type=jnp.float32)
        o_ref[g, :, :DIM] = y


def _tc_attention(shuf, w_t, bias2d, scale, gpb=4):
    n_groups = shuf.shape[0]
    return pl.pallas_call(
        functools.partial(_attn_body, scale, gpb),
        grid=(n_groups // gpb,),
        in_specs=[
            pl.BlockSpec((gpb, GROUP, QKV_PAD), lambda i: (i, 0, 0)),
            pl.BlockSpec((DIM, DIM), lambda i: (0, 0)),
            pl.BlockSpec((1, DIM), lambda i: (0, 0)),
        ],
        out_specs=pl.BlockSpec((gpb, GROUP, OUT_PAD), lambda i: (i, 0, 0)),
        out_shape=jax.ShapeDtypeStruct((n_groups, GROUP, OUT_PAD),
                                       jnp.float32),
    )(shuf, w_t, bias2d)


@functools.lru_cache(maxsize=None)
def _make_sc_row_scatter_multi(rows: int, d: int, chunk: int, parts: int):
    assert rows % (_NW * chunk * parts) == 0
    part_rows = rows // parts
    per_worker = part_rows // _NW
    n_chunks = per_worker // chunk
    mesh = plsc.VectorSubcoreMesh(core_axis_name="c", subcore_axis_name="s")

    @functools.partial(
        pl.kernel,
        out_type=jax.ShapeDtypeStruct((rows, d), jnp.float32),
        mesh=mesh,
        scratch_types=[
            pltpu.VMEM((chunk,), jnp.int32),
            pltpu.VMEM((chunk,), jnp.int32),
            pltpu.VMEM((chunk, d), jnp.float32),
            pltpu.VMEM((chunk, d), jnp.float32),
            pltpu.SemaphoreType.DMA,
            pltpu.SemaphoreType.DMA,
            pltpu.SemaphoreType.DMA,
            pltpu.SemaphoreType.DMA,
        ],
    )
    def scatter_kernel(*refs):
        srcs = refs[:parts]
        idx_hbm, out_hbm = refs[parts], refs[parts + 1]
        idx0, idx1, rows0, rows1, sl0, sl1, ss0, ss1 = refs[parts + 2:]
        wid = lax.axis_index("s") * _NC + lax.axis_index("c")
        bufs = [(idx0, rows0, sl0, ss0), (idx1, rows1, sl1, ss1)]
        scat_h = [None, None]
        i = 0
        for p in range(parts):
            base = wid * per_worker
            for ch in range(n_chunks):
                idx_v, rows_v, sl_, ss = bufs[i % 2]
                loc = base + ch * chunk
                if scat_h[i % 2] is not None:
                    scat_h[i % 2].wait()
                pltpu.sync_copy(idx_hbm.at[pl.ds(p * part_rows + loc, chunk)],
                                idx_v)
                pltpu.async_copy(srcs[p].at[pl.ds(loc, chunk)], rows_v,
                                 sl_).wait()
                scat_h[i % 2] = pltpu.async_copy(rows_v, out_hbm.at[idx_v], ss)
                i += 1
        scat_h[0].wait()
        scat_h[1].wait()

    return scatter_kernel


def kernel(qkv, tk_id, x_size, proj_w, proj_b):
    b, n, c3 = qkv.shape
    c = c3 // 3
    scale = (c // NUM_HEADS) ** (-0.5)
    ng = n // GROUP
    parts = 4
    part_rows = b * n // parts

    sort_idx = jnp.argsort(tk_id, axis=-1)

    offs = (jnp.arange(b, dtype=jnp.int32) * n)[:, None]
    sidx = (sort_idx.astype(jnp.int32) + offs).reshape(b * n)

    qkv_pad = jnp.pad(qkv.reshape(b * n, c3), ((0, 0), (0, QKV_PAD - c3)))

    gather = _make_sc_row_gather(part_rows, QKV_PAD, 64)
    wt = proj_w.T
    b2 = proj_b.reshape(1, c)
    ys = []
    for p in range(parts):
        sidx_p = lax.dynamic_slice_in_dim(sidx, p * part_rows, part_rows)
        shuf_p = gather(qkv_pad, sidx_p)
        y_p = _tc_attention(
            shuf_p.reshape(part_rows // GROUP, GROUP, QKV_PAD), wt, b2, scale)
        ys.append(y_p.reshape(part_rows, OUT_PAD))

    xp = _make_sc_row_scatter_multi(b * n, OUT_PAD, 128, parts)(*ys, sidx)
    return xp[:, :c].reshape(b, n, c)
